# baseline (speedup 1.0000x reference)
import functools

import jax
import jax.numpy as jnp
from jax import lax
from jax.experimental import pallas as pl
from jax.experimental.pallas import tpu as pltpu
from jax.experimental.pallas import tpu_sc as plsc

K = 16
EPS = 1e-5
N = 10000
C = 128
H = 64
NP = 10240
M1 = 256
M3 = 1000


S = 80
W = 128
R = 5


def _stage1_body(posb, post, xtb, w1t, wzt, idx_ref, y_ref, z_ref,
                 dist_ref, pv_ref, pg_ref):
    y_ref[...] = jnp.dot(xtb[...], w1t[...], preferred_element_type=jnp.float32)
    z_ref[...] = jnp.dot(xtb[...], wzt[...], preferred_element_type=jnp.float32)
    p = posb[...]
    pt = post[...]
    dots = jnp.dot(p, pt, preferred_element_type=jnp.float32)
    sqi = jnp.sum(p * p, axis=1, keepdims=True)
    sqj = jnp.sum(pt * pt, axis=0, keepdims=True)
    d2 = sqi + sqj - 2.0 * dots
    dist_ref[...] = jnp.sqrt(jnp.maximum(d2, 0.0))
    iota_w = lax.broadcasted_iota(jnp.int32, (M1, W), 1)
    for s in range(S):
        seg = dist_ref[:, s * W:(s + 1) * W]
        for r in range(R):
            mn = jnp.min(seg, axis=1, keepdims=True)
            sel = jnp.min(jnp.where(seg == mn, iota_w, W), axis=1,
                          keepdims=True)
            col = r * S + s
            pv_ref[:, col:col + 1] = mn
            pg_ref[:, col:col + 1] = sel + s * W
            seg = jnp.where(iota_w == sel, jnp.inf, seg)
    vpool = pv_ref[...]
    gpool = pg_ref[...]
    cols = []
    for _ in range(K):
        mn = jnp.min(vpool, axis=1, keepdims=True)
        eq = vpool == mn
        sel = jnp.min(jnp.where(eq, gpool, NP), axis=1, keepdims=True)
        cols.append(sel)
        vpool = jnp.where(eq & (gpool == sel), jnp.inf, vpool)
    idx_ref[...] = jnp.concatenate(cols, axis=1)
    drained = vpool[:, (R - 1) * S:] == jnp.inf
    bad = jnp.max(jnp.where(drained, 1, 0)) >= 1

    @pl.when(bad)
    def _fallback():
        iota = lax.broadcasted_iota(jnp.int32, (M1, NP), 1)
        cols2 = []
        for _ in range(K):
            dd = dist_ref[...]
            mn = jnp.min(dd, axis=1, keepdims=True)
            sel = jnp.min(jnp.where(dd == mn, iota, NP), axis=1, keepdims=True)
            cols2.append(sel)
            dist_ref[...] = jnp.where(iota == sel, jnp.inf, dd)
        idx_ref[...] = jnp.concatenate(cols2, axis=1)


def _stage1(posp, post, xtp, w1t, wzt):
    grid = NP // M1
    return pl.pallas_call(
        _stage1_body,
        grid=(grid,),
        in_specs=[
            pl.BlockSpec((M1, 8), lambda i: (i, 0)),
            pl.BlockSpec((8, NP), lambda i: (0, 0)),
            pl.BlockSpec((M1, C), lambda i: (i, 0)),
            pl.BlockSpec((C, H), lambda i: (0, 0)),
            pl.BlockSpec((C, H), lambda i: (0, 0)),
        ],
        out_specs=[
            pl.BlockSpec((M1, K), lambda i: (i, 0)),
            pl.BlockSpec((M1, H), lambda i: (i, 0)),
            pl.BlockSpec((M1, H), lambda i: (i, 0)),
        ],
        out_shape=[
            jax.ShapeDtypeStruct((NP, K), jnp.int32),
            jax.ShapeDtypeStruct((NP, H), jnp.float32),
            jax.ShapeDtypeStruct((NP, H), jnp.float32),
        ],
        scratch_shapes=[
            pltpu.VMEM((M1, NP), jnp.float32),
            pltpu.VMEM((M1, S * R), jnp.float32),
            pltpu.VMEM((M1, S * R), jnp.int32),
        ],
    )(posp, post, xtp, w1t, wzt)


def _stage3_body(g_ref, z_ref, xtb_ref, beta1_ref, attw_ref, fwt_ref,
                 beta2_ref, out_ref):
    zb = z_ref[...]
    beta1 = beta1_ref[...]
    attw = attw_ref[...]
    logits = []
    for k in range(K):
        h = jax.nn.relu(g_ref[k] + zb + beta1)
        logits.append(jnp.sum(h * attw, axis=1, keepdims=True))
    m = logits[0]
    for k in range(1, K):
        m = jnp.maximum(m, logits[k])
    num = jnp.zeros((M3, H), jnp.float32)
    den = jnp.zeros((M3, 1), jnp.float32)
    for k in range(K):
        e = jnp.exp(logits[k] - m)
        h = jax.nn.relu(g_ref[k] + zb + beta1)
        num = num + e * h
        den = den + e
    agg = num / den
    o = jnp.dot(agg, fwt_ref[...], preferred_element_type=jnp.float32)
    o = jax.nn.relu(o + beta2_ref[...])
    out_ref[...] = o + xtb_ref[...]


def _stage3(g, z, xt, beta1, attw, fwt, beta2):
    grid = N // M3
    return pl.pallas_call(
        _stage3_body,
        grid=(grid,),
        in_specs=[
            pl.BlockSpec((K, M3, H), lambda i: (0, i, 0)),
            pl.BlockSpec((M3, H), lambda i: (i, 0)),
            pl.BlockSpec((M3, C), lambda i: (i, 0)),
            pl.BlockSpec((1, H), lambda i: (0, 0)),
            pl.BlockSpec((1, H), lambda i: (0, 0)),
            pl.BlockSpec((H, C), lambda i: (0, 0)),
            pl.BlockSpec((1, C), lambda i: (0, 0)),
        ],
        out_specs=pl.BlockSpec((M3, C), lambda i: (i, 0)),
        out_shape=jax.ShapeDtypeStruct((N, C), jnp.float32),
    )(g, z, xt, beta1, attw, fwt, beta2)


NW = 32
ROWS_PER_W = K * N // NW
NCH = 125
CH = ROWS_PER_W // NCH


def _sc_gather(y, idx3d):
    mesh = plsc.VectorSubcoreMesh(core_axis_name="c", subcore_axis_name="s")

    @functools.partial(
        pl.kernel,
        out_type=jax.ShapeDtypeStruct((K * N, H), jnp.float32),
        mesh=mesh,
        scratch_types=[
            pltpu.VMEM((NCH, CH), jnp.int32),
            pltpu.VMEM((CH, H), jnp.float32),
            pltpu.SemaphoreType.DMA,
        ],
        compiler_params=pltpu.CompilerParams(use_tc_tiling_on_sc=False),
    )
    def gk(y_hbm, idx_hbm, out_hbm, idx_v, rows_v, sem):
        wid = lax.axis_index("s") * 2 + lax.axis_index("c")
        pltpu.sync_copy(idx_hbm.at[wid], idx_v)

        def body(c, carry):
            pltpu.async_copy(y_hbm.at[idx_v.at[c]], rows_v, sem).wait()
            pltpu.sync_copy(rows_v,
                            out_hbm.at[pl.ds(wid * ROWS_PER_W + c * CH, CH)])
            return carry

        lax.fori_loop(0, NCH, body, 0)

    return gk(y, idx3d)


def kernel(x, pos, edge_conv_W, edge_bn_gamma, edge_bn_beta, att_W, fuse_W,
           fuse_bn_gamma, fuse_bn_beta):
    xt = x[0].T
    xtp = jnp.pad(xt, ((0, NP - N), (0, 0)))
    posp = jnp.zeros((NP, 8), jnp.float32)
    posp = posp.at[:N, :3].set(pos[0])
    posp = posp.at[N:, 0].set(1e6)
    post = posp.T

    scale1 = edge_bn_gamma / jnp.sqrt(1.0 + EPS)
    w1 = edge_conv_W[:, :C]
    w2 = edge_conv_W[:, C:]
    w1t = (w1 * scale1[:, None]).T
    wzt = ((w2 - w1) * scale1[:, None]).T
    scale2 = fuse_bn_gamma / jnp.sqrt(1.0 + EPS)
    fwt = (fuse_W * scale2[:, None]).T
    beta1 = edge_bn_beta[None, :]
    beta2 = fuse_bn_beta[None, :]

    idx, y, z = _stage1(posp, post, xtp, w1t, wzt)
    idx3d = idx[:N].T.reshape(NW, NCH, CH)
    g = _sc_gather(y, idx3d).reshape(K, N, H)
    out = _stage3(g, z[:N], xt, beta1, att_W, fwt, beta2)
    return out.T[None]


# speedup vs baseline: 2.0050x; 2.0050x over previous
import functools

import jax
import jax.numpy as jnp
from jax import lax
from jax.experimental import pallas as pl
from jax.experimental.pallas import tpu as pltpu
from jax.experimental.pallas import tpu_sc as plsc

K = 16
EPS = 1e-5
N = 10000
C = 128
H = 64
NP = 10240
M1 = 256
M3 = 1000


def _stage1_body(posb, post, xtb, w1t, wzt, idx_ref, y_ref, z_ref, dist_ref):
    y_ref[...] = jnp.dot(xtb[...], w1t[...], preferred_element_type=jnp.float32)
    z_ref[...] = jnp.dot(xtb[...], wzt[...], preferred_element_type=jnp.float32)
    p = posb[...]
    pt = post[...]
    dots = jnp.dot(p, pt, preferred_element_type=jnp.float32)
    sqi = jnp.sum(p * p, axis=1, keepdims=True)
    sqj = jnp.sum(pt * pt, axis=0, keepdims=True)
    d2 = sqi + sqj - 2.0 * dots
    dist_ref[...] = jnp.sqrt(jnp.maximum(d2, 0.0))
    iota = lax.broadcasted_iota(jnp.int32, (M1, NP), 1)
    cols = []
    for _ in range(K):
        d = dist_ref[...]
        mn = jnp.min(d, axis=1, keepdims=True)
        sel = jnp.min(jnp.where(d == mn, iota, NP), axis=1, keepdims=True)
        cols.append(sel)
        dist_ref[...] = jnp.where(iota == sel, jnp.inf, d)
    idx_ref[...] = jnp.concatenate(cols, axis=1)


def _stage1(posp, post, xtp, w1t, wzt):
    grid = NP // M1
    return pl.pallas_call(
        _stage1_body,
        grid=(grid,),
        in_specs=[
            pl.BlockSpec((M1, 8), lambda i: (i, 0)),
            pl.BlockSpec((8, NP), lambda i: (0, 0)),
            pl.BlockSpec((M1, C), lambda i: (i, 0)),
            pl.BlockSpec((C, H), lambda i: (0, 0)),
            pl.BlockSpec((C, H), lambda i: (0, 0)),
        ],
        out_specs=[
            pl.BlockSpec((M1, K), lambda i: (i, 0)),
            pl.BlockSpec((M1, H), lambda i: (i, 0)),
            pl.BlockSpec((M1, H), lambda i: (i, 0)),
        ],
        out_shape=[
            jax.ShapeDtypeStruct((NP, K), jnp.int32),
            jax.ShapeDtypeStruct((NP, H), jnp.float32),
            jax.ShapeDtypeStruct((NP, H), jnp.float32),
        ],
        scratch_shapes=[pltpu.VMEM((M1, NP), jnp.float32)],
    )(posp, post, xtp, w1t, wzt)


def _stage3_body(g_ref, z_ref, xtb_ref, beta1_ref, attw_ref, fwt_ref,
                 beta2_ref, out_ref):
    zb = z_ref[...]
    beta1 = beta1_ref[...]
    attw = attw_ref[...]
    logits = []
    for k in range(K):
        h = jax.nn.relu(g_ref[k] + zb + beta1)
        logits.append(jnp.sum(h * attw, axis=1, keepdims=True))
    m = logits[0]
    for k in range(1, K):
        m = jnp.maximum(m, logits[k])
    num = jnp.zeros((M3, H), jnp.float32)
    den = jnp.zeros((M3, 1), jnp.float32)
    for k in range(K):
        e = jnp.exp(logits[k] - m)
        h = jax.nn.relu(g_ref[k] + zb + beta1)
        num = num + e * h
        den = den + e
    agg = num / den
    o = jnp.dot(agg, fwt_ref[...], preferred_element_type=jnp.float32)
    o = jax.nn.relu(o + beta2_ref[...])
    out_ref[...] = o + xtb_ref[...]


def _stage3(g, z, xt, beta1, attw, fwt, beta2):
    grid = N // M3
    return pl.pallas_call(
        _stage3_body,
        grid=(grid,),
        in_specs=[
            pl.BlockSpec((K, M3, H), lambda i: (0, i, 0)),
            pl.BlockSpec((M3, H), lambda i: (i, 0)),
            pl.BlockSpec((M3, C), lambda i: (i, 0)),
            pl.BlockSpec((1, H), lambda i: (0, 0)),
            pl.BlockSpec((1, H), lambda i: (0, 0)),
            pl.BlockSpec((H, C), lambda i: (0, 0)),
            pl.BlockSpec((1, C), lambda i: (0, 0)),
        ],
        out_specs=pl.BlockSpec((M3, C), lambda i: (i, 0)),
        out_shape=jax.ShapeDtypeStruct((N, C), jnp.float32),
    )(g, z, xt, beta1, attw, fwt, beta2)


NW = 32
ROWS_PER_W = K * N // NW
NCH = 125
CH = ROWS_PER_W // NCH


def _sc_gather(y, idx3d):
    mesh = plsc.VectorSubcoreMesh(core_axis_name="c", subcore_axis_name="s")

    @functools.partial(
        pl.kernel,
        out_type=jax.ShapeDtypeStruct((K * N, H), jnp.float32),
        mesh=mesh,
        scratch_types=[
            pltpu.VMEM((NCH, CH), jnp.int32),
            pltpu.VMEM((CH, H), jnp.float32),
            pltpu.SemaphoreType.DMA,
        ],
        compiler_params=pltpu.CompilerParams(use_tc_tiling_on_sc=False),
    )
    def gk(y_hbm, idx_hbm, out_hbm, idx_v, rows_v, sem):
        wid = lax.axis_index("s") * 2 + lax.axis_index("c")
        pltpu.sync_copy(idx_hbm.at[wid], idx_v)

        def body(c, carry):
            pltpu.async_copy(y_hbm.at[idx_v.at[c]], rows_v, sem).wait()
            pltpu.sync_copy(rows_v,
                            out_hbm.at[pl.ds(wid * ROWS_PER_W + c * CH, CH)])
            return carry

        lax.fori_loop(0, NCH, body, 0)

    return gk(y, idx3d)


def kernel(x, pos, edge_conv_W, edge_bn_gamma, edge_bn_beta, att_W, fuse_W,
           fuse_bn_gamma, fuse_bn_beta):
    xt = x[0].T
    xtp = jnp.pad(xt, ((0, NP - N), (0, 0)))
    posp = jnp.zeros((NP, 8), jnp.float32)
    posp = posp.at[:N, :3].set(pos[0])
    posp = posp.at[N:, 0].set(1e6)
    post = posp.T

    scale1 = edge_bn_gamma / jnp.sqrt(1.0 + EPS)
    w1 = edge_conv_W[:, :C]
    w2 = edge_conv_W[:, C:]
    w1t = (w1 * scale1[:, None]).T
    wzt = ((w2 - w1) * scale1[:, None]).T
    scale2 = fuse_bn_gamma / jnp.sqrt(1.0 + EPS)
    fwt = (fuse_W * scale2[:, None]).T
    beta1 = edge_bn_beta[None, :]
    beta2 = fuse_bn_beta[None, :]

    idx, y, z = _stage1(posp, post, xtp, w1t, wzt)
    idx3d = idx[:N].T.reshape(NW, NCH, CH)
    g = _sc_gather(y, idx3d).reshape(K, N, H)
    out = _stage3(g, z[:N], xt, beta1, att_W, fwt, beta2)
    return out.T[None]


# speedup vs baseline: 2.0346x; 1.0148x over previous
import functools

import jax
import jax.numpy as jnp
from jax import lax
from jax.experimental import pallas as pl
from jax.experimental.pallas import tpu as pltpu
from jax.experimental.pallas import tpu_sc as plsc

K = 16
EPS = 1e-5
N = 10000
C = 128
H = 64
NP = 10240
NH = NP // 2
M1 = 256
M3 = 1000


def _stage1_body(posb, post, xtb, w1t, wzt, idx_ref, y_ref, z_ref,
                 e_ref, o_ref, w_ref, v_ref):
    y_ref[...] = jnp.dot(xtb[...], w1t[...], preferred_element_type=jnp.float32)
    z_ref[...] = jnp.dot(xtb[...], wzt[...], preferred_element_type=jnp.float32)
    p = posb[...]
    pt = post[...]
    dots = jnp.dot(p, pt, preferred_element_type=jnp.float32)
    sqi = jnp.sum(p * p, axis=1, keepdims=True)
    sqj = jnp.sum(pt * pt, axis=0, keepdims=True)
    d2 = sqi + sqj - 2.0 * dots
    dist = jnp.sqrt(jnp.maximum(d2, 0.0))
    lo = dist[:, :NH]
    hi = dist[:, NH:]
    c = lo <= hi
    iota_h = lax.broadcasted_iota(jnp.int32, (M1, NH), 1)
    e_ref[...] = jnp.where(c, lo, hi)
    o_ref[...] = jnp.where(c, hi, lo)
    w_ref[...] = jnp.where(c, iota_h, iota_h + NH)
    v_ref[...] = jnp.where(c, iota_h + NH, iota_h)
    cols = []
    for _ in range(K):
        e = e_ref[...]
        w = w_ref[...]
        mn = jnp.min(e, axis=1, keepdims=True)
        widx = jnp.min(jnp.where(e == mn, w, NP), axis=1, keepdims=True)
        cols.append(widx)
        pm = w == widx
        v = v_ref[...]
        refill = jnp.where(w == v, jnp.inf, o_ref[...])
        e_ref[...] = jnp.where(pm, refill, e)
        w_ref[...] = jnp.where(pm, v, w)
    idx_ref[...] = jnp.concatenate(cols, axis=1)


def _stage1(posp, post, xtp, w1t, wzt):
    grid = NP // M1
    return pl.pallas_call(
        _stage1_body,
        grid=(grid,),
        in_specs=[
            pl.BlockSpec((M1, 8), lambda i: (i, 0)),
            pl.BlockSpec((8, NP), lambda i: (0, 0)),
            pl.BlockSpec((M1, C), lambda i: (i, 0)),
            pl.BlockSpec((C, H), lambda i: (0, 0)),
            pl.BlockSpec((C, H), lambda i: (0, 0)),
        ],
        out_specs=[
            pl.BlockSpec((M1, K), lambda i: (i, 0)),
            pl.BlockSpec((M1, H), lambda i: (i, 0)),
            pl.BlockSpec((M1, H), lambda i: (i, 0)),
        ],
        out_shape=[
            jax.ShapeDtypeStruct((NP, K), jnp.int32),
            jax.ShapeDtypeStruct((NP, H), jnp.float32),
            jax.ShapeDtypeStruct((NP, H), jnp.float32),
        ],
        scratch_shapes=[
            pltpu.VMEM((M1, NH), jnp.float32),
            pltpu.VMEM((M1, NH), jnp.float32),
            pltpu.VMEM((M1, NH), jnp.int32),
            pltpu.VMEM((M1, NH), jnp.int32),
        ],
    )(posp, post, xtp, w1t, wzt)


def _stage3_body(g_ref, z_ref, xtb_ref, beta1_ref, attw_ref, fwt_ref,
                 beta2_ref, out_ref):
    zb = z_ref[...]
    beta1 = beta1_ref[...]
    attw = attw_ref[...]
    logits = []
    for k in range(K):
        h = jax.nn.relu(g_ref[k] + zb + beta1)
        logits.append(jnp.sum(h * attw, axis=1, keepdims=True))
    m = logits[0]
    for k in range(1, K):
        m = jnp.maximum(m, logits[k])
    num = jnp.zeros((M3, H), jnp.float32)
    den = jnp.zeros((M3, 1), jnp.float32)
    for k in range(K):
        e = jnp.exp(logits[k] - m)
        h = jax.nn.relu(g_ref[k] + zb + beta1)
        num = num + e * h
        den = den + e
    agg = num / den
    o = jnp.dot(agg, fwt_ref[...], preferred_element_type=jnp.float32)
    o = jax.nn.relu(o + beta2_ref[...])
    out_ref[...] = o + xtb_ref[...]


def _stage3(g, z, xt, beta1, attw, fwt, beta2):
    grid = N // M3
    return pl.pallas_call(
        _stage3_body,
        grid=(grid,),
        in_specs=[
            pl.BlockSpec((K, M3, H), lambda i: (0, i, 0)),
            pl.BlockSpec((M3, H), lambda i: (i, 0)),
            pl.BlockSpec((M3, C), lambda i: (i, 0)),
            pl.BlockSpec((1, H), lambda i: (0, 0)),
            pl.BlockSpec((1, H), lambda i: (0, 0)),
            pl.BlockSpec((H, C), lambda i: (0, 0)),
            pl.BlockSpec((1, C), lambda i: (0, 0)),
        ],
        out_specs=pl.BlockSpec((M3, C), lambda i: (i, 0)),
        out_shape=jax.ShapeDtypeStruct((N, C), jnp.float32),
    )(g, z, xt, beta1, attw, fwt, beta2)


NW = 32
ROWS_PER_W = K * N // NW
NCH = 125
CH = ROWS_PER_W // NCH


def _sc_gather(y, idx3d):
    mesh = plsc.VectorSubcoreMesh(core_axis_name="c", subcore_axis_name="s")

    @functools.partial(
        pl.kernel,
        out_type=jax.ShapeDtypeStruct((K * N, H), jnp.float32),
        mesh=mesh,
        scratch_types=[
            pltpu.VMEM((NCH, CH), jnp.int32),
            pltpu.VMEM((CH, H), jnp.float32),
            pltpu.SemaphoreType.DMA,
        ],
        compiler_params=pltpu.CompilerParams(use_tc_tiling_on_sc=False),
    )
    def gk(y_hbm, idx_hbm, out_hbm, idx_v, rows_v, sem):
        wid = lax.axis_index("s") * 2 + lax.axis_index("c")
        pltpu.sync_copy(idx_hbm.at[wid], idx_v)

        def body(c, carry):
            pltpu.async_copy(y_hbm.at[idx_v.at[c]], rows_v, sem).wait()
            pltpu.sync_copy(rows_v,
                            out_hbm.at[pl.ds(wid * ROWS_PER_W + c * CH, CH)])
            return carry

        lax.fori_loop(0, NCH, body, 0)

    return gk(y, idx3d)


def kernel(x, pos, edge_conv_W, edge_bn_gamma, edge_bn_beta, att_W, fuse_W,
           fuse_bn_gamma, fuse_bn_beta):
    xt = x[0].T
    xtp = jnp.pad(xt, ((0, NP - N), (0, 0)))
    posp = jnp.zeros((NP, 8), jnp.float32)
    posp = posp.at[:N, :3].set(pos[0])
    posp = posp.at[N:, 0].set(1e6)
    post = posp.T

    scale1 = edge_bn_gamma / jnp.sqrt(1.0 + EPS)
    w1 = edge_conv_W[:, :C]
    w2 = edge_conv_W[:, C:]
    w1t = (w1 * scale1[:, None]).T
    wzt = ((w2 - w1) * scale1[:, None]).T
    scale2 = fuse_bn_gamma / jnp.sqrt(1.0 + EPS)
    fwt = (fuse_W * scale2[:, None]).T
    beta1 = edge_bn_beta[None, :]
    beta2 = fuse_bn_beta[None, :]

    idx, y, z = _stage1(posp, post, xtp, w1t, wzt)
    idx3d = idx[:N].T.reshape(NW, NCH, CH)
    g = _sc_gather(y, idx3d).reshape(K, N, H)
    out = _stage3(g, z[:N], xt, beta1, att_W, fwt, beta2)
    return out.T[None]


# speedup vs baseline: 2.0949x; 1.0296x over previous
import functools

import jax
import jax.numpy as jnp
from jax import lax
from jax.experimental import pallas as pl
from jax.experimental.pallas import tpu as pltpu
from jax.experimental.pallas import tpu_sc as plsc

K = 16
EPS = 1e-5
N = 10000
C = 128
H = 64
NP = 10240
NH = NP // 2
M1 = 256
M3 = 1000


def _stage1_body(posb, post, xtb, w1t, wzt, idx_ref, y_ref, z_ref,
                 e_ref, o_ref, w_ref, v_ref):
    y_ref[...] = jnp.dot(xtb[...], w1t[...], preferred_element_type=jnp.float32)
    z_ref[...] = jnp.dot(xtb[...], wzt[...], preferred_element_type=jnp.float32)
    p = posb[...]
    pt = post[...]
    dots = jnp.dot(p, pt, preferred_element_type=jnp.float32)
    sqi = jnp.sum(p * p, axis=1, keepdims=True)
    sqj = jnp.sum(pt * pt, axis=0, keepdims=True)
    d2 = sqi + sqj - 2.0 * dots
    dist = jnp.sqrt(jnp.maximum(d2, 0.0))
    lo = dist[:, :NH]
    hi = dist[:, NH:]
    c = lo <= hi
    iota_h = lax.broadcasted_iota(jnp.int32, (M1, NH), 1)
    e_ref[...] = jnp.where(c, lo, hi)
    o_ref[...] = jnp.where(c, hi, lo)
    w_ref[...] = jnp.where(c, iota_h, iota_h + NH)
    v_ref[...] = jnp.where(c, iota_h + NH, iota_h)
    cols = []
    for _ in range(K):
        e = e_ref[...]
        w = w_ref[...]
        mn = jnp.min(e, axis=1, keepdims=True)
        widx = jnp.min(jnp.where(e == mn, w, NP), axis=1, keepdims=True)
        cols.append(widx)
        pm = w == widx
        v = v_ref[...]
        refill = jnp.where(w == v, jnp.inf, o_ref[...])
        e_ref[...] = jnp.where(pm, refill, e)
        w_ref[...] = jnp.where(pm, v, w)
    idx_ref[...] = jnp.concatenate(cols, axis=1)


def _stage1(posp, post, xtp, w1t, wzt):
    grid = NP // M1
    return pl.pallas_call(
        _stage1_body,
        grid=(grid,),
        in_specs=[
            pl.BlockSpec((M1, 8), lambda i: (i, 0)),
            pl.BlockSpec((8, NP), lambda i: (0, 0)),
            pl.BlockSpec((M1, C), lambda i: (i, 0)),
            pl.BlockSpec((C, H), lambda i: (0, 0)),
            pl.BlockSpec((C, H), lambda i: (0, 0)),
        ],
        out_specs=[
            pl.BlockSpec((M1, K), lambda i: (i, 0)),
            pl.BlockSpec((M1, H), lambda i: (i, 0)),
            pl.BlockSpec((M1, H), lambda i: (i, 0)),
        ],
        out_shape=[
            jax.ShapeDtypeStruct((NP, K), jnp.int32),
            jax.ShapeDtypeStruct((NP, H), jnp.float32),
            jax.ShapeDtypeStruct((NP, H), jnp.float32),
        ],
        scratch_shapes=[
            pltpu.VMEM((M1, NH), jnp.float32),
            pltpu.VMEM((M1, NH), jnp.float32),
            pltpu.VMEM((M1, NH), jnp.int32),
            pltpu.VMEM((M1, NH), jnp.int32),
        ],
    )(posp, post, xtp, w1t, wzt)


def _stage3_body(g_ref, z_ref, xtb_ref, beta1_ref, attw_ref, fwt_ref,
                 beta2_ref, out_ref):
    zb = z_ref[...]
    beta1 = beta1_ref[...]
    attw = attw_ref[...]
    logits = []
    for k in range(K):
        h = jax.nn.relu(g_ref[k] + zb + beta1)
        logits.append(jnp.sum(h * attw, axis=1, keepdims=True))
    m = logits[0]
    for k in range(1, K):
        m = jnp.maximum(m, logits[k])
    num = jnp.zeros((M3, H), jnp.float32)
    den = jnp.zeros((M3, 1), jnp.float32)
    for k in range(K):
        e = jnp.exp(logits[k] - m)
        h = jax.nn.relu(g_ref[k] + zb + beta1)
        num = num + e * h
        den = den + e
    agg = num / den
    o = jnp.dot(agg, fwt_ref[...], preferred_element_type=jnp.float32)
    o = jax.nn.relu(o + beta2_ref[...])
    out_ref[...] = o + xtb_ref[...]


def _stage3(g, z, xt, beta1, attw, fwt, beta2):
    grid = N // M3
    return pl.pallas_call(
        _stage3_body,
        grid=(grid,),
        in_specs=[
            pl.BlockSpec((K, M3, H), lambda i: (0, i, 0)),
            pl.BlockSpec((M3, H), lambda i: (i, 0)),
            pl.BlockSpec((M3, C), lambda i: (i, 0)),
            pl.BlockSpec((1, H), lambda i: (0, 0)),
            pl.BlockSpec((1, H), lambda i: (0, 0)),
            pl.BlockSpec((H, C), lambda i: (0, 0)),
            pl.BlockSpec((1, C), lambda i: (0, 0)),
        ],
        out_specs=pl.BlockSpec((M3, C), lambda i: (i, 0)),
        out_shape=jax.ShapeDtypeStruct((N, C), jnp.float32),
    )(g, z, xt, beta1, attw, fwt, beta2)


NW = 32
ROWS_PER_W = K * N // NW
NCH = 125
CH = ROWS_PER_W // NCH
G = 5
NG = NCH // G


def _sc_gather(y, idx3d):
    mesh = plsc.VectorSubcoreMesh(core_axis_name="c", subcore_axis_name="s")

    @functools.partial(
        pl.kernel,
        out_type=jax.ShapeDtypeStruct((K * N, H), jnp.float32),
        mesh=mesh,
        scratch_types=[
            pltpu.VMEM((NCH, CH), jnp.int32),
            pltpu.VMEM((G * CH, H), jnp.float32),
            pltpu.SemaphoreType.DMA,
        ],
        compiler_params=pltpu.CompilerParams(use_tc_tiling_on_sc=False),
    )
    def gk(y_hbm, idx_hbm, out_hbm, idx_v, rows_v, sem):
        wid = lax.axis_index("s") * 2 + lax.axis_index("c")
        pltpu.sync_copy(idx_hbm.at[wid], idx_v)

        def body(g, carry):
            descs = [
                pltpu.async_copy(y_hbm.at[idx_v.at[g * G + j]],
                                 rows_v.at[pl.ds(j * CH, CH)], sem)
                for j in range(G)
            ]
            for d in descs:
                d.wait()
            pltpu.sync_copy(
                rows_v,
                out_hbm.at[pl.ds(wid * ROWS_PER_W + g * (G * CH), G * CH)])
            return carry

        lax.fori_loop(0, NG, body, 0)

    return gk(y, idx3d)


def kernel(x, pos, edge_conv_W, edge_bn_gamma, edge_bn_beta, att_W, fuse_W,
           fuse_bn_gamma, fuse_bn_beta):
    xt = x[0].T
    xtp = jnp.pad(xt, ((0, NP - N), (0, 0)))
    posp = jnp.zeros((NP, 8), jnp.float32)
    posp = posp.at[:N, :3].set(pos[0])
    posp = posp.at[N:, 0].set(1e6)
    post = posp.T

    scale1 = edge_bn_gamma / jnp.sqrt(1.0 + EPS)
    w1 = edge_conv_W[:, :C]
    w2 = edge_conv_W[:, C:]
    w1t = (w1 * scale1[:, None]).T
    wzt = ((w2 - w1) * scale1[:, None]).T
    scale2 = fuse_bn_gamma / jnp.sqrt(1.0 + EPS)
    fwt = (fuse_W * scale2[:, None]).T
    beta1 = edge_bn_beta[None, :]
    beta2 = fuse_bn_beta[None, :]

    idx, y, z = _stage1(posp, post, xtp, w1t, wzt)
    idx3d = idx[:N].T.reshape(NW, NCH, CH)
    g = _sc_gather(y, idx3d).reshape(K, N, H)
    out = _stage3(g, z[:N], xt, beta1, att_W, fwt, beta2)
    return out.T[None]
